# Initial kernel scaffold; baseline (speedup 1.0000x reference)
import jax
import jax.numpy as jnp
from jax import lax
from jax.experimental import pallas as pl
from jax.experimental.pallas import tpu as pltpu
from jax.experimental.pallas import tpu_sc as plsc

MAXV = 100000
D = 32
HW = 256
NIDX = 3 * HW
NC, NS = 2, 16
NW = NC * NS
B = 1024
IMGS_PER_W = B // NW


def _sc_body(inputs_hbm, table_hbm, out_hbm, idx_v, rows_v, out_v, sem):
    wid = lax.axis_index("s") * NC + lax.axis_index("c")
    iota = lax.iota(jnp.int32, 16)

    @pl.loop(0, IMGS_PER_W)
    def _img(k):
        img = wid * IMGS_PER_W + k

        pltpu.sync_copy(inputs_hbm.at[img], idx_v)

        for r in range(6):
            off = jnp.int32((r // 2) * MAXV)

            @pl.loop(0, 8)
            def _off(i):
                sl = pl.ds(i * 16, 16)
                idx_v[r, sl] = idx_v[r, sl] + off

        copies = [
            pltpu.async_copy(
                table_hbm.at[idx_v.at[j]],
                rows_v.at[pl.ds(j * 128, 128)],
                sem,
            )
            for j in range(6)
        ]
        for c in copies:
            c.wait()

        @pl.loop(0, 512)
        def _acc(t):
            d = t >> 4
            p = ((t & 15) * 16) + iota
            dvec = jnp.full((16,), d, jnp.int32)
            e0 = plsc.load_gather(rows_v, [p, dvec])
            e1 = plsc.load_gather(rows_v, [p + 256, dvec])
            e2 = plsc.load_gather(rows_v, [p + 512, dvec])
            out_v[pl.ds(t * 16, 16)] = e0 + e1 + e2

        pltpu.sync_copy(out_v, out_hbm.at[img])


@jax.jit
def _bow_embed(inputs_flat, table):
    f = pl.kernel(
        _sc_body,
        out_type=jax.ShapeDtypeStruct((B, D * HW), jnp.float32),
        mesh=plsc.VectorSubcoreMesh(core_axis_name="c", subcore_axis_name="s"),
        scratch_types=[
            pltpu.VMEM((6, 128), jnp.int32),
            pltpu.VMEM((NIDX, D), jnp.float32),
            pltpu.VMEM((D * HW,), jnp.float32),
            pltpu.SemaphoreType.DMA,
        ],
    )
    return f(inputs_flat, table)


def kernel(inputs, table):
    inputs_flat = inputs.reshape(B, 6, 128)
    out = _bow_embed(inputs_flat, table)
    return out.reshape(B, D, 16, 16)


# speedup vs baseline: 28.9968x; 28.9968x over previous
import jax
import jax.numpy as jnp
from jax import lax
from jax.experimental import pallas as pl
from jax.experimental.pallas import tpu as pltpu
from jax.experimental.pallas import tpu_sc as plsc

MAXV = 100000
D = 32
HW = 256
NIDX = 3 * HW
NC, NS = 2, 16
NW = NC * NS
B = 1024
IMGS_PER_W = B // NW


def _sc_body(inputs_hbm, table_hbm, out_hbm, idx_v, rows_v, out_v, sem):
    wid = lax.axis_index("s") * NC + lax.axis_index("c")
    iota = lax.iota(jnp.int32, 16)

    @pl.loop(0, IMGS_PER_W)
    def _img(k):
        img = wid * IMGS_PER_W + k

        pltpu.sync_copy(inputs_hbm.at[img], idx_v)

        for r in range(6):
            off = jnp.int32((r // 2) * MAXV)

            @pl.loop(0, 8)
            def _off(i):
                sl = pl.ds(i * 16, 16)
                idx_v[r, sl] = idx_v[r, sl] + off

        copies = [
            pltpu.async_copy(
                table_hbm.at[idx_v.at[j]],
                rows_v.at[pl.ds(j * 128, 128)],
                sem,
            )
            for j in range(6)
        ]
        for c in copies:
            c.wait()

        lo = iota * 256
        hi = lo + 16 * 256

        @pl.loop(0, HW)
        def _acc(p):
            s0 = pl.ds(0, 16)
            s1 = pl.ds(16, 16)
            a0 = rows_v[p, s0] + rows_v[p + 256, s0] + rows_v[p + 512, s0]
            a1 = rows_v[p, s1] + rows_v[p + 256, s1] + rows_v[p + 512, s1]
            plsc.store_scatter(out_v, [p + lo], a0)
            plsc.store_scatter(out_v, [p + hi], a1)

        pltpu.sync_copy(out_v, out_hbm.at[img])


@jax.jit
def _bow_embed(inputs_flat, table):
    f = pl.kernel(
        _sc_body,
        out_type=jax.ShapeDtypeStruct((B, D * HW), jnp.float32),
        mesh=plsc.VectorSubcoreMesh(core_axis_name="c", subcore_axis_name="s"),
        compiler_params=pltpu.CompilerParams(
            needs_layout_passes=False, use_tc_tiling_on_sc=False
        ),
        scratch_types=[
            pltpu.VMEM((6, 128), jnp.int32),
            pltpu.VMEM((NIDX, D), jnp.float32),
            pltpu.VMEM((D * HW,), jnp.float32),
            pltpu.SemaphoreType.DMA,
        ],
    )
    return f(inputs_flat, table)


def kernel(inputs, table):
    inputs_flat = inputs.reshape(B, 6, 128)
    out = _bow_embed(inputs_flat, table)
    return out.reshape(B, D, 16, 16)


# speedup vs baseline: 33.2286x; 1.1459x over previous
import jax
import jax.numpy as jnp
from jax import lax
from jax.experimental import pallas as pl
from jax.experimental.pallas import tpu as pltpu
from jax.experimental.pallas import tpu_sc as plsc

MAXV = 100000
D = 32
HW = 256
NIDX = 3 * HW
NC, NS = 2, 16
NW = NC * NS
B = 1024
IMGS_PER_W = B // NW


def _sc_body(inputs_hbm, table_hbm, out_hbm, idx_v, rows_v, out_v, gsems, osem):
    wid = lax.axis_index("s") * NC + lax.axis_index("c")
    iota = lax.iota(jnp.int32, 16)
    base = wid * IMGS_PER_W

    def stage(par, img):
        pltpu.sync_copy(inputs_hbm.at[img], idx_v.at[par])
        for r in range(6):
            off = jnp.int32((r // 2) * MAXV)

            @pl.loop(0, 8)
            def _off(i):
                sl = pl.ds(i * 16, 16)
                idx_v[par, r, sl] = idx_v[par, r, sl] + off

        for j in range(6):
            pltpu.async_copy(
                table_hbm.at[idx_v.at[par, j]],
                rows_v.at[par, pl.ds(j * 128, 128)],
                gsems[par],
            )

    def wait_gathers(par):
        for j in range(6):
            pltpu.make_async_copy(
                table_hbm.at[idx_v.at[par, j]],
                rows_v.at[par, pl.ds(j * 128, 128)],
                gsems[par],
            ).wait()

    stage(0, base)

    @pl.loop(0, IMGS_PER_W // 2)
    def _pair(kk):
        for par in range(2):
            k = kk * 2 + par
            img = base + k

            @pl.when(k + 1 < IMGS_PER_W)
            def _():
                stage(1 - par, img + 1)

            wait_gathers(par)

            @pl.when(k >= 2)
            def _():
                pltpu.make_async_copy(
                    out_v.at[par], out_hbm.at[img - 2], osem
                ).wait()

            lo = iota * 256
            hi = lo + 16 * 256

            @pl.loop(0, HW)
            def _acc(p):
                s0 = pl.ds(0, 16)
                s1 = pl.ds(16, 16)
                a0 = (
                    rows_v[par, p, s0]
                    + rows_v[par, p + 256, s0]
                    + rows_v[par, p + 512, s0]
                )
                a1 = (
                    rows_v[par, p, s1]
                    + rows_v[par, p + 256, s1]
                    + rows_v[par, p + 512, s1]
                )
                plsc.store_scatter(out_v.at[par], [p + lo], a0)
                plsc.store_scatter(out_v.at[par], [p + hi], a1)

            pltpu.async_copy(out_v.at[par], out_hbm.at[img], osem)

    for par in range(2):
        img = base + IMGS_PER_W - 2 + par
        pltpu.make_async_copy(out_v.at[par], out_hbm.at[img], osem).wait()


@jax.jit
def _bow_embed(inputs_flat, table):
    f = pl.kernel(
        _sc_body,
        out_type=jax.ShapeDtypeStruct((B, D * HW), jnp.float32),
        mesh=plsc.VectorSubcoreMesh(core_axis_name="c", subcore_axis_name="s"),
        compiler_params=pltpu.CompilerParams(
            needs_layout_passes=False, use_tc_tiling_on_sc=False
        ),
        scratch_types=[
            pltpu.VMEM((2, 6, 128), jnp.int32),
            pltpu.VMEM((2, NIDX, D), jnp.float32),
            pltpu.VMEM((2, D * HW), jnp.float32),
            [pltpu.SemaphoreType.DMA, pltpu.SemaphoreType.DMA],
            pltpu.SemaphoreType.DMA,
        ],
    )
    return f(inputs_flat, table)


def kernel(inputs, table):
    inputs_flat = inputs.reshape(B, 6, 128)
    out = _bow_embed(inputs_flat, table)
    return out.reshape(B, D, 16, 16)
